# baseline (speedup 1.0000x reference)
import jax
import jax.numpy as jnp
from jax.experimental import pallas as pl
from jax.experimental.pallas import tpu as pltpu

H, W_ = 180, 360
HW = H * W_
_TH = 32
_NHB = 6
_FULL = _TH * W_
_TSTART = 57600
_TSIZE = 7168
_TOFF = 5 * _FULL - _TSTART
_TROWS = 20


def _fold_body(x_hbm, xt_ref, o_ref, buf, sems):
    n = pl.program_id(0)
    hb = pl.program_id(1)
    s = n * _NHB + hb
    slot = s % 2
    last = pl.num_programs(0) * _NHB - 1

    def in_op(step, slt, do_start):
        sn = step // _NHB
        shb = step - sn * _NHB

        @pl.when(shb < _NHB - 1)
        def _full():
            for k in range(2):
                cp = pltpu.make_async_copy(
                    x_hbm.at[sn, pl.ds(k * 96, 96),
                             pl.ds(pl.multiple_of(shb * _FULL, 128), _FULL)],
                    buf.at[slt, pl.ds(k * 96, 96), pl.ds(0, _FULL)],
                    sems.at[slt, k],
                )
                cp.start() if do_start else cp.wait()

        @pl.when(shb == _NHB - 1)
        def _tail():
            for k in range(2):
                cp = pltpu.make_async_copy(
                    x_hbm.at[sn, pl.ds(k * 96, 96), pl.ds(_TSTART, _TSIZE)],
                    buf.at[slt, pl.ds(k * 96, 96), pl.ds(0, _TSIZE)],
                    sems.at[slt, k],
                )
                cp.start() if do_start else cp.wait()

    @pl.when(s == 0)
    def _prologue():
        in_op(0, 0, True)

    @pl.when(s < last)
    def _prefetch():
        in_op(s + 1, slot ^ 1, True)

    in_op(s, slot, False)

    @pl.when(hb < _NHB - 1)
    def _emit_full():
        for t in range(_TH):
            o_ref[0, t] = buf[slot, :, t * W_:(t + 1) * W_]

    @pl.when(hb == _NHB - 1)
    def _emit_tail():
        tl = _TROWS - 1
        for t in range(tl):
            o_ref[0, t] = buf[slot, :, _TOFF + t * W_:_TOFF + (t + 1) * W_]
        o_ref[0, tl, :, 0:_TSIZE - _TOFF - tl * W_] = (
            buf[slot, :, _TOFF + tl * W_:_TSIZE])
        o_ref[0, tl, :, _TSIZE - _TOFF - tl * W_:W_] = xt_ref[0]


def kernel(x):
    N, C, L = x.shape
    xt = x[:, :, _TSTART + _TSIZE:]
    out = pl.pallas_call(
        _fold_body,
        grid=(N, _NHB),
        in_specs=[
            pl.BlockSpec(memory_space=pl.ANY),
            pl.BlockSpec((1, C, L - _TSTART - _TSIZE), lambda n, h: (n, 0, 0)),
        ],
        out_specs=pl.BlockSpec((1, _TH, C, W_), lambda n, h: (n, h, 0, 0)),
        out_shape=jax.ShapeDtypeStruct((N, H, C, W_), x.dtype),
        scratch_shapes=[
            pltpu.VMEM((2, C, _FULL), jnp.float32),
            pltpu.SemaphoreType.DMA((2, 2)),
        ],
    )(x, xt)
    return out.transpose(0, 2, 1, 3)


# speedup vs baseline: 1.0005x; 1.0005x over previous
import jax
import jax.numpy as jnp
from jax.experimental import pallas as pl
from jax.experimental.pallas import tpu as pltpu

H, W_ = 180, 360
HW = H * W_
_TH = 32
_NHB = 6
_FULL = _TH * W_
_TSTART = 57600
_TSIZE = 7168
_TOFF = 5 * _FULL - _TSTART
_TROWS = 20


def _fold_body(x_hbm, xt_ref, o_ref, buf, sems):
    n = pl.program_id(0)
    hb = pl.program_id(1)
    s = n * _NHB + hb
    slot = s % 2
    last = pl.num_programs(0) * _NHB - 1

    def in_op(step, slt, do_start):
        sn = step // _NHB
        shb = step - sn * _NHB

        @pl.when(shb < _NHB - 1)
        def _full():
            cp = pltpu.make_async_copy(
                x_hbm.at[sn, :, pl.ds(pl.multiple_of(shb * _FULL, 128), _FULL)],
                buf.at[slt, :, pl.ds(0, _FULL)],
                sems.at[slt],
            )
            cp.start() if do_start else cp.wait()

        @pl.when(shb == _NHB - 1)
        def _tail():
            cp = pltpu.make_async_copy(
                x_hbm.at[sn, :, pl.ds(_TSTART, _TSIZE)],
                buf.at[slt, :, pl.ds(0, _TSIZE)],
                sems.at[slt],
            )
            cp.start() if do_start else cp.wait()

    @pl.when(s == 0)
    def _prologue():
        in_op(0, 0, True)

    @pl.when(s < last)
    def _prefetch():
        in_op(s + 1, slot ^ 1, True)

    in_op(s, slot, False)

    @pl.when(hb < _NHB - 1)
    def _emit_full():
        for t in range(_TH):
            o_ref[0, t] = buf[slot, :, t * W_:(t + 1) * W_]

    @pl.when(hb == _NHB - 1)
    def _emit_tail():
        tl = _TROWS - 1
        for t in range(tl):
            o_ref[0, t] = buf[slot, :, _TOFF + t * W_:_TOFF + (t + 1) * W_]
        o_ref[0, tl, :, 0:_TSIZE - _TOFF - tl * W_] = (
            buf[slot, :, _TOFF + tl * W_:_TSIZE])
        o_ref[0, tl, :, _TSIZE - _TOFF - tl * W_:W_] = xt_ref[0]


def kernel(x):
    N, C, L = x.shape
    xt = x[:, :, _TSTART + _TSIZE:]
    out = pl.pallas_call(
        _fold_body,
        grid=(N, _NHB),
        in_specs=[
            pl.BlockSpec(memory_space=pl.ANY),
            pl.BlockSpec((1, C, L - _TSTART - _TSIZE), lambda n, h: (n, 0, 0)),
        ],
        out_specs=pl.BlockSpec((1, _TH, C, W_), lambda n, h: (n, h, 0, 0)),
        out_shape=jax.ShapeDtypeStruct((N, H, C, W_), x.dtype),
        scratch_shapes=[
            pltpu.VMEM((2, C, _FULL), jnp.float32),
            pltpu.SemaphoreType.DMA((2,)),
        ],
    )(x, xt)
    return out.transpose(0, 2, 1, 3)
